# Initial kernel scaffold; baseline (speedup 1.0000x reference)
import jax
import jax.numpy as jnp
from jax.experimental import pallas as pl


def kernel(data, adj, W, bias):
    raise NotImplementedError("write your pallas kernel here")


# speedup vs baseline: 1.4448x; 1.4448x over previous
import jax
import jax.numpy as jnp
from jax.experimental import pallas as pl

B, N, F_IN, F_OUT, K = 4, 2048, 256, 256, 3
_DIAG_BLK = 256


def _cheb_kernel(data_ref, adj_ref, w_ref, bias_ref, out_ref):
    adj = adj_ref[:]
    x = data_ref[:].reshape(B * N, F_IN)
    x0 = x[:N]

    rowsum = jnp.sum(adj, axis=1, keepdims=True)
    diag_parts = []
    for i in range(N // _DIAG_BLK):
        blk = adj[i * _DIAG_BLK:(i + 1) * _DIAG_BLK, :]
        r = jax.lax.broadcasted_iota(jnp.int32, (_DIAG_BLK, N), 0) + i * _DIAG_BLK
        c = jax.lax.broadcasted_iota(jnp.int32, (_DIAG_BLK, N), 1)
        diag_parts.append(
            jnp.sum(jnp.where(r == c, blk, 0.0), axis=1, keepdims=True))
    diagv = jnp.concatenate(diag_parts, axis=0)

    deg = rowsum - diagv
    dinv = jnp.where(deg > 0, jax.lax.rsqrt(jnp.where(deg > 0, deg, 1.0)), 0.0)

    def s_apply(y):
        ys = dinv * y
        m = jax.lax.dot_general(adj, ys, (((0,), (0,)), ((), ())),
                                preferred_element_type=jnp.float32)
        return -dinv * (m - diagv * ys)

    t1 = s_apply(x0)
    t2 = s_apply(t1)

    w0 = w_ref[0]
    w1 = w_ref[1]
    w2 = w_ref[2]
    wc = w0 - w2

    out = jnp.dot(x, wc, preferred_element_type=jnp.float32) + bias_ref[:]
    extra = (jnp.dot(t1, w1, preferred_element_type=jnp.float32)
             + jnp.dot(2.0 * t2, w2, preferred_element_type=jnp.float32))
    out_ref[:] = out
    out_ref[:N, :] += extra


def kernel(data, adj, W, bias):
    out = pl.pallas_call(
        _cheb_kernel,
        out_shape=jax.ShapeDtypeStruct((B * N, F_OUT), jnp.float32),
    )(data, adj, W, bias.reshape(1, F_OUT))
    return out.reshape(B, N, F_OUT)


# speedup vs baseline: 1.4560x; 1.0078x over previous
import jax
import jax.numpy as jnp
from jax.experimental import pallas as pl

B, N, F_IN, F_OUT, K = 4, 2048, 256, 256, 3
_DIAG_BLK = 256


def _cheb_kernel(data_ref, adj_ref, w_ref, bias_ref, out_ref):
    adj = adj_ref[:]
    x = data_ref[:].reshape(B * N, F_IN)
    x0 = x[:N]

    rowsum = jnp.sum(adj, axis=1, keepdims=True)
    diag_parts = []
    for i in range(N // _DIAG_BLK):
        blk = adj[i * _DIAG_BLK:(i + 1) * _DIAG_BLK, :]
        r = jax.lax.broadcasted_iota(jnp.int32, (_DIAG_BLK, N), 0) + i * _DIAG_BLK
        c = jax.lax.broadcasted_iota(jnp.int32, (_DIAG_BLK, N), 1)
        diag_parts.append(
            jnp.sum(jnp.where(r == c, blk, 0.0), axis=1, keepdims=True))
    diagv = jnp.concatenate(diag_parts, axis=0)

    deg = rowsum - diagv
    dinv = jnp.where(deg > 0, jax.lax.rsqrt(jnp.where(deg > 0, deg, 1.0)), 0.0)

    adj_bf = adj.astype(jnp.bfloat16)

    def s_apply(y):
        ys = dinv * y
        m = jax.lax.dot_general(adj_bf, ys.astype(jnp.bfloat16),
                                (((0,), (0,)), ((), ())),
                                preferred_element_type=jnp.float32)
        return -dinv * (m - diagv * ys)

    t1 = s_apply(x0)
    t2 = s_apply(t1)

    w0 = w_ref[0]
    w1 = w_ref[1]
    w2 = w_ref[2]
    wc = (w0 - w2).astype(jnp.bfloat16)

    out = jnp.dot(x.astype(jnp.bfloat16), wc,
                  preferred_element_type=jnp.float32) + bias_ref[:]
    extra = (jnp.dot(t1.astype(jnp.bfloat16), w1.astype(jnp.bfloat16),
                     preferred_element_type=jnp.float32)
             + jnp.dot((2.0 * t2).astype(jnp.bfloat16), w2.astype(jnp.bfloat16),
                       preferred_element_type=jnp.float32))
    out_ref[:] = out
    out_ref[:N, :] += extra


def kernel(data, adj, W, bias):
    out = pl.pallas_call(
        _cheb_kernel,
        out_shape=jax.ShapeDtypeStruct((B * N, F_OUT), jnp.float32),
    )(data, adj, W, bias.reshape(1, F_OUT))
    return out.reshape(B, N, F_OUT)
